# Initial kernel scaffold; baseline (speedup 1.0000x reference)
import jax
import jax.numpy as jnp
from jax.experimental import pallas as pl


def kernel(word_ids, cand_ids, word_table, ent_table, W, b):
    raise NotImplementedError("write your pallas kernel here")


# speedup vs baseline: 3.9589x; 3.9589x over previous
import functools

import jax
import jax.numpy as jnp
from jax import lax
from jax.experimental import pallas as pl
from jax.experimental.pallas import tpu as pltpu
from jax.experimental.pallas import tpu_sc as plsc

B, L, C, D = 4096, 200, 20, 64
CP = 32
NC, NS, LANES = 2, 16, 16
NW = NC * NS
RPW = B // NW
G1 = 128
G2 = L - G1


def _body(wids, cids, wtab, etab, wt, bias, out,
          widx, cidx, wrows, crows, wtv, bv, scoresv, tmat, sem):
    wid = lax.axis_index("s") * NC + lax.axis_index("c")
    base = wid * RPW

    pltpu.sync_copy(wids.at[pl.ds(base, RPW)], widx)
    pltpu.sync_copy(cids.at[pl.ds(base, RPW)], cidx)
    pltpu.sync_copy(wt, wtv)
    pltpu.sync_copy(bias, bv)

    lane = lax.iota(jnp.int32, LANES)

    def row_body(j, _):
        cp1 = pltpu.async_copy(wtab.at[widx.at[j, pl.ds(0, G1)]],
                               wrows.at[pl.ds(0, G1)], sem)
        cp2 = pltpu.async_copy(wtab.at[widx.at[j, pl.ds(G1, G2)]],
                               wrows.at[pl.ds(G1, G2)], sem)
        cp3 = pltpu.async_copy(etab.at[cidx.at[j]], crows, sem)

        one = jnp.ones((LANES,), jnp.float32)
        zf = jnp.zeros((LANES,), jnp.float32)
        z16 = jnp.zeros((LANES,), jnp.int32)
        cntv = zf
        for k in range(L // LANES):
            v = widx[j, pl.ds(k * LANES, LANES)]
            cntv = cntv + jnp.where(v != 0, one, zf)
        vt = widx[j, pl.ds(L - LANES, LANES)]
        cntv = cntv + jnp.where(
            (vt != 0) & (lane >= LANES - (L % LANES)), one, zf)
        tmat[0, pl.ds(0, LANES)] = cntv
        t0 = zf
        t1 = zf
        for k2 in range(0, LANES, 2):
            t0 = t0 + plsc.load_gather(
                tmat, [z16, jnp.full((LANES,), k2, jnp.int32)])
            t1 = t1 + plsc.load_gather(
                tmat, [z16, jnp.full((LANES,), k2 + 1, jnp.int32)])
        inv = one / jnp.maximum(t0 + t1, one)

        cp1.wait()
        cp2.wait()
        cp3.wait()

        def sum_body(li, accs):
            a = list(accs)
            for u in range(8):
                l = li * 8 + u
                for k in range(4):
                    r = wrows[l, pl.ds(k * LANES, LANES)]
                    i = k + 4 * (u % 2)
                    a[i] = a[i] + r
            return tuple(a)

        accs = lax.fori_loop(0, L // 8, sum_body, (zf,) * 8)
        pooled = [(accs[k] + accs[k + 4]) * inv for k in range(4)]

        pacc = [bv[pl.ds(k * LANES, LANES)] for k in range(4)]
        for kin in range(4):
            p = pooled[kin]
            for u in range(LANES):
                s = p[u]
                for k in range(4):
                    pacc[k] = pacc[k] + wtv[kin * LANES + u, pl.ds(k * LANES, LANES)] * s

        def tbuild(c, cc):
            t = crows[c, pl.ds(0, LANES)] * pacc[0]
            for k in range(1, 4):
                t = t + crows[c, pl.ds(k * LANES, LANES)] * pacc[k]
            tmat[cc, pl.ds(0, LANES)] = t

        def tsum():
            r0 = zf
            r1 = zf
            for k2 in range(0, LANES, 2):
                c0 = plsc.load_gather(
                    tmat, [lane, jnp.full((LANES,), k2, jnp.int32)])
                c1 = plsc.load_gather(
                    tmat, [lane, jnp.full((LANES,), k2 + 1, jnp.int32)])
                r0 = r0 + c0
                r1 = r1 + c1
            return r0 + r1

        for cc in range(LANES):
            tbuild(cc, cc)
        scoresv[j, pl.ds(0, LANES)] = tsum()
        for cc in range(C - LANES):
            tbuild(LANES + cc, cc)
        for cc in range(C - LANES, LANES):
            tmat[cc, pl.ds(0, LANES)] = zf
        scoresv[j, pl.ds(LANES, LANES)] = tsum()
        return 0

    lax.fori_loop(0, RPW, row_body, 0)
    pltpu.sync_copy(scoresv, out.at[pl.ds(base, RPW)])


@functools.partial(
    pl.kernel,
    out_type=jax.ShapeDtypeStruct((B, CP), jnp.float32),
    mesh=plsc.VectorSubcoreMesh(core_axis_name="c", subcore_axis_name="s"),
    compiler_params=pltpu.CompilerParams(
        needs_layout_passes=False, use_tc_tiling_on_sc=False),
    scratch_types=[
        pltpu.VMEM((RPW, L), jnp.int32),
        pltpu.VMEM((RPW, C), jnp.int32),
        pltpu.VMEM((L, D), jnp.float32),
        pltpu.VMEM((C, D), jnp.float32),
        pltpu.VMEM((D, D), jnp.float32),
        pltpu.VMEM((D,), jnp.float32),
        pltpu.VMEM((RPW, CP), jnp.float32),
        pltpu.VMEM((LANES, LANES), jnp.float32),
        pltpu.SemaphoreType.DMA,
    ],
)
def _yamada_sc(*refs):
    _body(*refs)


def kernel(word_ids, cand_ids, word_table, ent_table, W, b):
    wids = word_ids.astype(jnp.int32)
    cids = cand_ids.astype(jnp.int32)
    wt = jnp.transpose(W)
    return _yamada_sc(wids, cids, word_table, ent_table, wt, b)[:, :C]


# speedup vs baseline: 4.5266x; 1.1434x over previous
import functools

import jax
import jax.numpy as jnp
from jax import lax
from jax.experimental import pallas as pl
from jax.experimental.pallas import tpu as pltpu
from jax.experimental.pallas import tpu_sc as plsc

B, L, C, D = 4096, 200, 20, 64
CP = 32
NC, NS, LANES = 2, 16, 16
NW = NC * NS
RPW = B // NW
G1 = 128
G2 = L - G1


def _body(wids, cids, wtab, etab, wt, bias, out,
          widx, cidx, wrows0, crows0, wrows1, crows1,
          wtv, bv, scoresv, tmat, sem0, sem1):
    wid = lax.axis_index("s") * NC + lax.axis_index("c")
    base = wid * RPW

    pltpu.sync_copy(wids.at[pl.ds(base, RPW)], widx)
    pltpu.sync_copy(cids.at[pl.ds(base, RPW)], cidx)
    pltpu.sync_copy(wt, wtv)
    pltpu.sync_copy(bias, bv)

    lane = lax.iota(jnp.int32, LANES)

    def issue(j, wrows, crows, sem):
        pltpu.async_copy(wtab.at[widx.at[j, pl.ds(0, G1)]],
                         wrows.at[pl.ds(0, G1)], sem)
        pltpu.async_copy(wtab.at[widx.at[j, pl.ds(G1, G2)]],
                         wrows.at[pl.ds(G1, G2)], sem)
        pltpu.async_copy(etab.at[cidx.at[j]], crows, sem)

    def drain(wrows, crows, sem):
        pltpu.make_async_copy(wtab.at[pl.ds(0, G1)],
                              wrows.at[pl.ds(0, G1)], sem).wait()
        pltpu.make_async_copy(wtab.at[pl.ds(0, G2)],
                              wrows.at[pl.ds(G1, G2)], sem).wait()
        pltpu.make_async_copy(etab.at[pl.ds(0, C)], crows, sem).wait()

    def compute(j, wrows, crows):
        one = jnp.ones((LANES,), jnp.float32)
        zf = jnp.zeros((LANES,), jnp.float32)
        z16 = jnp.zeros((LANES,), jnp.int32)
        cntv = zf
        for k in range(L // LANES):
            v = widx[j, pl.ds(k * LANES, LANES)]
            cntv = cntv + jnp.where(v != 0, one, zf)
        vt = widx[j, pl.ds(L - LANES, LANES)]
        cntv = cntv + jnp.where(
            (vt != 0) & (lane >= LANES - (L % LANES)), one, zf)
        tmat[0, pl.ds(0, LANES)] = cntv
        t0 = zf
        t1 = zf
        for k2 in range(0, LANES, 2):
            t0 = t0 + plsc.load_gather(
                tmat, [z16, jnp.full((LANES,), k2, jnp.int32)])
            t1 = t1 + plsc.load_gather(
                tmat, [z16, jnp.full((LANES,), k2 + 1, jnp.int32)])
        inv = one / jnp.maximum(t0 + t1, one)

        def sum_body(li, accs):
            a = list(accs)
            for u in range(8):
                l = li * 8 + u
                for k in range(4):
                    r = wrows[l, pl.ds(k * LANES, LANES)]
                    i = k + 4 * (u % 2)
                    a[i] = a[i] + r
            return tuple(a)

        accs = lax.fori_loop(0, L // 8, sum_body, (zf,) * 8)
        pooled = [(accs[k] + accs[k + 4]) * inv for k in range(4)]

        pacc = [bv[pl.ds(k * LANES, LANES)] for k in range(4)]
        for kin in range(4):
            p = pooled[kin]
            for u in range(LANES):
                s = p[u]
                for k in range(4):
                    pacc[k] = pacc[k] + wtv[kin * LANES + u, pl.ds(k * LANES, LANES)] * s

        def tbuild(c, cc):
            t = crows[c, pl.ds(0, LANES)] * pacc[0]
            for k in range(1, 4):
                t = t + crows[c, pl.ds(k * LANES, LANES)] * pacc[k]
            tmat[cc, pl.ds(0, LANES)] = t

        def tsum():
            r0 = zf
            r1 = zf
            for k2 in range(0, LANES, 2):
                c0 = plsc.load_gather(
                    tmat, [lane, jnp.full((LANES,), k2, jnp.int32)])
                c1 = plsc.load_gather(
                    tmat, [lane, jnp.full((LANES,), k2 + 1, jnp.int32)])
                r0 = r0 + c0
                r1 = r1 + c1
            return r0 + r1

        for cc in range(LANES):
            tbuild(cc, cc)
        scoresv[j, pl.ds(0, LANES)] = tsum()
        for cc in range(C - LANES):
            tbuild(LANES + cc, cc)
        for cc in range(C - LANES, LANES):
            tmat[cc, pl.ds(0, LANES)] = zf
        scoresv[j, pl.ds(LANES, LANES)] = tsum()

    issue(0, wrows0, crows0, sem0)
    issue(1, wrows1, crows1, sem1)

    def pair_body(i, _):
        b0 = 2 * i
        drain(wrows0, crows0, sem0)
        compute(b0, wrows0, crows0)
        issue(jnp.minimum(b0 + 2, RPW - 1), wrows0, crows0, sem0)
        drain(wrows1, crows1, sem1)
        compute(b0 + 1, wrows1, crows1)
        issue(jnp.minimum(b0 + 3, RPW - 1), wrows1, crows1, sem1)
        return 0

    lax.fori_loop(0, RPW // 2, pair_body, 0)
    drain(wrows0, crows0, sem0)
    drain(wrows1, crows1, sem1)
    pltpu.sync_copy(scoresv, out.at[pl.ds(base, RPW)])


@functools.partial(
    pl.kernel,
    out_type=jax.ShapeDtypeStruct((B, CP), jnp.float32),
    mesh=plsc.VectorSubcoreMesh(core_axis_name="c", subcore_axis_name="s"),
    compiler_params=pltpu.CompilerParams(
        needs_layout_passes=False, use_tc_tiling_on_sc=False),
    scratch_types=[
        pltpu.VMEM((RPW, L), jnp.int32),
        pltpu.VMEM((RPW, C), jnp.int32),
        pltpu.VMEM((L, D), jnp.float32),
        pltpu.VMEM((C, D), jnp.float32),
        pltpu.VMEM((L, D), jnp.float32),
        pltpu.VMEM((C, D), jnp.float32),
        pltpu.VMEM((D, D), jnp.float32),
        pltpu.VMEM((D,), jnp.float32),
        pltpu.VMEM((RPW, CP), jnp.float32),
        pltpu.VMEM((LANES, LANES), jnp.float32),
        pltpu.SemaphoreType.DMA,
        pltpu.SemaphoreType.DMA,
    ],
)
def _yamada_sc(*refs):
    _body(*refs)


def kernel(word_ids, cand_ids, word_table, ent_table, W, b):
    wids = word_ids.astype(jnp.int32)
    cids = cand_ids.astype(jnp.int32)
    wt = jnp.transpose(W)
    return _yamada_sc(wids, cids, word_table, ent_table, wt, b)[:, :C]
